# Initial kernel scaffold; baseline (speedup 1.0000x reference)
import jax
import jax.numpy as jnp
from jax.experimental import pallas as pl


def kernel(x, edge_index, adj_vals, deg, sigma1, W1, b1, sigma2, W2, b2):
    raise NotImplementedError("write your pallas kernel here")


# speedup vs baseline: 4.1435x; 4.1435x over previous
import functools

import jax
import jax.numpy as jnp
from jax import lax
from jax.experimental import pallas as pl
from jax.experimental.pallas import tpu as pltpu
from jax.experimental.pallas import tpu_sc as plsc

_NC = 2
_NS = 16
_L = 16
_CH = 128


def _zero_chunk_rows(rpt):
    for z in range(min(rpt, 64), 0, -1):
        if rpt % z == 0:
            return z
    return 1


def _lane_bcast(v16, lane):
    sel = jnp.full((_L,), lane, jnp.int32)
    return v16.at[sel].get(mode="promise_in_bounds")


def _scale_groups(av_ref, n, blockfn):
    gfull, rem = n // _L, n % _L

    def group(g, en):
        av16 = av_ref[pl.ds(g * _L, _L)]
        for e16 in range(en):
            blockfn(g * _L + e16, _lane_bcast(av16, e16))

    if gfull:
        def gbody(g, carry):
            group(g, _L)
            return carry
        lax.fori_loop(0, gfull, gbody, 0)
    if rem:
        group(gfull, rem)


def _sc_mesh():
    return plsc.VectorSubcoreMesh(core_axis_name="c", subcore_axis_name="s")


def _edge_mask_sc(x, src, dst, adj):
    N, F = x.shape
    E = src.shape[0]
    EC = E // _NC
    ET = EC // _NS
    nfull = ET // _CH
    tail = ET % _CH
    NP = -(-N // (_NS * 128)) * (_NS * 128)
    RPT = NP // _NS
    ZB = _zero_chunk_rows(RPT)
    nf = F // _L

    scratch = [
        pltpu.VMEM_SHARED((NP, F), jnp.float32),
        pltpu.VMEM((_CH,), jnp.int32),
        pltpu.VMEM((_CH,), jnp.int32),
        pltpu.VMEM((_CH,), jnp.float32),
        pltpu.VMEM((_CH, F), jnp.float32),
        pltpu.VMEM((_CH, F), jnp.float32),
        pltpu.VMEM((ZB, F), jnp.float32),
    ]
    if tail:
        tpad = ((tail + _L - 1) // _L) * _L
        scratch += [
            pltpu.VMEM((tail,), jnp.int32),
            pltpu.VMEM((tail,), jnp.int32),
            pltpu.VMEM((tpad,), jnp.float32),
            pltpu.VMEM((tail, F), jnp.float32),
            pltpu.VMEM((tail, F), jnp.float32),
        ]

    @functools.partial(
        pl.kernel,
        out_type=jax.ShapeDtypeStruct((_NC * NP, F), jnp.float32),
        mesh=_sc_mesh(),
        scratch_types=scratch,
        compiler_params=pltpu.CompilerParams(use_tc_tiling_on_sc=False),
    )
    def body(x_hbm, src_hbm, dst_hbm, adj_hbm, out_hbm, acc,
             src_v, dst_v, adj_v, rs, rd, zbuf, *tl):
        cid = lax.axis_index("c")
        sid = lax.axis_index("s")
        zero16 = jnp.zeros((_L,), jnp.float32)

        def zrow(r, carry):
            for j in range(nf):
                zbuf[r, pl.ds(j * _L, _L)] = zero16
            return carry
        lax.fori_loop(0, ZB, zrow, 0)
        for k in range(RPT // ZB):
            pltpu.sync_copy(zbuf, acc.at[pl.ds(sid * RPT + k * ZB, ZB)])
        plsc.subcore_barrier()

        ebase = cid * EC + sid * ET

        def chunk(off, sv, dv, av, rsv, rdv, n):
            pltpu.sync_copy(src_hbm.at[pl.ds(off, n)], sv)
            pltpu.sync_copy(dst_hbm.at[pl.ds(off, n)], dv)
            pltpu.sync_copy(adj_hbm.at[pl.ds(off, n)], av.at[pl.ds(0, n)])
            pltpu.sync_copy(x_hbm.at[sv], rsv)
            pltpu.sync_copy(x_hbm.at[dv], rdv)

            def blockfn(e, avec):
                for j in range(nf):
                    sl = pl.ds(j * _L, _L)
                    d = rsv[e, sl] - rdv[e, sl]
                    rsv[e, sl] = d * d * avec
            _scale_groups(av, n, blockfn)
            pltpu.sync_copy(rsv, acc.at[sv], add=True)

        def main_loop(i, carry):
            chunk(ebase + i * _CH, src_v, dst_v, adj_v, rs, rd, _CH)
            return carry
        lax.fori_loop(0, nfull, main_loop, 0)
        if tail:
            chunk(ebase + nfull * _CH, tl[0], tl[1], tl[2], tl[3], tl[4], tail)
        plsc.subcore_barrier()

        out_base = cid * NP + sid * RPT
        for k in range(RPT // ZB):
            pltpu.sync_copy(acc.at[pl.ds(sid * RPT + k * ZB, ZB)], zbuf)
            pltpu.sync_copy(zbuf, out_hbm.at[pl.ds(out_base + k * ZB, ZB)])

    return body(x, src, dst, adj)


def _propagate_sc(y, src, dst, adj):
    N, G = y.shape
    E = src.shape[0]
    EC = E // _NC
    ET = EC // _NS
    nfull = ET // _CH
    tail = ET % _CH
    NP = -(-N // (_NS * 128)) * (_NS * 128)
    RPT = NP // _NS
    ZB = _zero_chunk_rows(RPT)
    ng = G // _L

    scratch = [
        pltpu.VMEM_SHARED((NP, G), jnp.float32),
        pltpu.VMEM((_CH,), jnp.int32),
        pltpu.VMEM((_CH,), jnp.int32),
        pltpu.VMEM((_CH,), jnp.float32),
        pltpu.VMEM((_CH, G), jnp.float32),
        pltpu.VMEM((ZB, G), jnp.float32),
    ]
    if tail:
        tpad = ((tail + _L - 1) // _L) * _L
        scratch += [
            pltpu.VMEM((tail,), jnp.int32),
            pltpu.VMEM((tail,), jnp.int32),
            pltpu.VMEM((tpad,), jnp.float32),
            pltpu.VMEM((tail, G), jnp.float32),
        ]

    @functools.partial(
        pl.kernel,
        out_type=jax.ShapeDtypeStruct((_NC * NP, G), jnp.float32),
        mesh=_sc_mesh(),
        scratch_types=scratch,
        compiler_params=pltpu.CompilerParams(use_tc_tiling_on_sc=False),
    )
    def body(y_hbm, src_hbm, dst_hbm, adj_hbm, out_hbm, acc,
             src_v, dst_v, adj_v, rows, zbuf, *tl):
        cid = lax.axis_index("c")
        sid = lax.axis_index("s")
        zero16 = jnp.zeros((_L,), jnp.float32)

        def zrow(r, carry):
            for j in range(ng):
                zbuf[r, pl.ds(j * _L, _L)] = zero16
            return carry
        lax.fori_loop(0, ZB, zrow, 0)
        for k in range(RPT // ZB):
            pltpu.sync_copy(zbuf, acc.at[pl.ds(sid * RPT + k * ZB, ZB)])
        plsc.subcore_barrier()

        ebase = cid * EC + sid * ET

        def chunk(off, sv, dv, av, rv, n):
            pltpu.sync_copy(src_hbm.at[pl.ds(off, n)], sv)
            pltpu.sync_copy(dst_hbm.at[pl.ds(off, n)], dv)
            pltpu.sync_copy(adj_hbm.at[pl.ds(off, n)], av.at[pl.ds(0, n)])
            pltpu.sync_copy(y_hbm.at[dv], rv)

            def blockfn(e, avec):
                for j in range(ng):
                    sl = pl.ds(j * _L, _L)
                    rv[e, sl] = rv[e, sl] * avec
            _scale_groups(av, n, blockfn)
            pltpu.sync_copy(rv, acc.at[sv], add=True)

        def main_loop(i, carry):
            chunk(ebase + i * _CH, src_v, dst_v, adj_v, rows, _CH)
            return carry
        lax.fori_loop(0, nfull, main_loop, 0)
        if tail:
            chunk(ebase + nfull * _CH, tl[0], tl[1], tl[2], tl[3], tail)
        plsc.subcore_barrier()

        out_base = cid * NP + sid * RPT
        for k in range(RPT // ZB):
            pltpu.sync_copy(acc.at[pl.ds(sid * RPT + k * ZB, ZB)], zbuf)
            pltpu.sync_copy(zbuf, out_hbm.at[pl.ds(out_base + k * ZB, ZB)])

    return body(y, src, dst, adj)


def _fc_tc(x, m0, m1, degcol, sigma, W, b):
    N, F = x.shape
    G = W.shape[1]

    def body(x_ref, m0_ref, m1_ref, deg_ref, sig_ref, w_ref, b_ref, o_ref):
        sig = sig_ref[...]
        inv = 1.0 / (sig * sig)
        t = (m0_ref[...] + m1_ref[...]) * inv / deg_ref[...]
        xm = jnp.exp(-t) * x_ref[...]
        o_ref[...] = (
            jnp.dot(xm, w_ref[...], preferred_element_type=jnp.float32)
            + b_ref[...]
        )

    return pl.pallas_call(
        body, out_shape=jax.ShapeDtypeStruct((N, G), jnp.float32),
    )(x, m0, m1, degcol, sigma.reshape(1, F), W, b.reshape(1, G))


def _relu_combine_tc(p0, p1):
    def body(a_ref, b_ref, o_ref):
        o_ref[...] = jnp.maximum(a_ref[...] + b_ref[...], 0.0)

    return pl.pallas_call(
        body, out_shape=jax.ShapeDtypeStruct(p0.shape, jnp.float32),
    )(p0, p1)


def _log_softmax_tc(p0, p1, nclass):
    N, GP = p0.shape

    def body(a_ref, b_ref, o_ref):
        s = a_ref[...] + b_ref[...]
        col = lax.broadcasted_iota(jnp.int32, (N, GP), 1)
        valid = col < nclass
        s = jnp.where(valid, s, -1e30)
        m = jnp.max(s, axis=1, keepdims=True)
        e = jnp.where(valid, jnp.exp(s - m), 0.0)
        lse = jnp.log(jnp.sum(e, axis=1, keepdims=True))
        r = s - m - lse
        o_ref[...] = r[:, :nclass]

    return pl.pallas_call(
        body, out_shape=jax.ShapeDtypeStruct((N, nclass), jnp.float32),
    )(p0, p1)


def kernel(x, edge_index, adj_vals, deg, sigma1, W1, b1, sigma2, W2, b2):
    N, F = x.shape
    src = edge_index[0]
    dst = edge_index[1]
    degcol = deg.reshape(N, 1)
    nclass = W2.shape[1]
    gp = ((nclass + _L - 1) // _L) * _L
    W2p = jnp.pad(W2, ((0, 0), (0, gp - nclass)))
    b2p = jnp.pad(b2, (0, gp - nclass))
    NP = -(-N // (_NS * 128)) * (_NS * 128)

    m1p = _edge_mask_sc(x, src, dst, adj_vals)
    y1 = _fc_tc(x, m1p[:N], m1p[NP:NP + N], degcol, sigma1, W1, b1)
    o1p = _propagate_sc(y1, src, dst, adj_vals)
    h = _relu_combine_tc(o1p[:N], o1p[NP:NP + N])
    m2p = _edge_mask_sc(h, src, dst, adj_vals)
    y2 = _fc_tc(h, m2p[:N], m2p[NP:NP + N], degcol, sigma2, W2p, b2p)
    o2p = _propagate_sc(y2, src, dst, adj_vals)
    return _log_softmax_tc(o2p[:N], o2p[NP:NP + N], nclass)
